# baseline (speedup 1.0000x reference)
import jax
import jax.numpy as jnp
from jax import lax
from jax.experimental import pallas as pl
from jax.experimental.pallas import tpu as pltpu
from jax.experimental.pallas import tpu_sc as plsc

N = 10000
E = 320000
WPAD = 16
NC, NS = 2, 16
NW = NC * NS
BATCH = 128
NROWS = E // BATCH
NBQ = NROWS // NW
NBX = NROWS - NW * NBQ
NB = NBQ + 2
NBUF = 4
MAINB = 76
NPAD = N + 112
RPS = NPAD // NS

_mesh = plsc.VectorSubcoreMesh(core_axis_name="c", subcore_axis_name="s",
                               num_cores=NC, num_subcores=NS)


def _fill(ref, n, val):
    def body(i, _):
        ref[i] = jnp.full((WPAD,), val, jnp.float32)
        return 0
    lax.fori_loop(0, n, body, 0)


def _zero_idx_row(ref, r):
    for k in range(BATCH // 16):
        ref[r, pl.ds(k * 16, 16)] = jnp.zeros((16,), jnp.int32)


def _load_batches(idx_hbm, idx_v, w):
    pltpu.sync_copy(idx_hbm.at[pl.ds(w * NBQ, NBQ)], idx_v.at[pl.ds(0, NBQ)])

    @pl.when(w < NBX)
    def _():
        pltpu.sync_copy(idx_hbm.at[pl.ds(NW * NBQ + w, 1)],
                        idx_v.at[pl.ds(NBQ, 1)])


def _deg_body(dst_hbm, out_hbm, idx_v, ones_v, zbuf_v, acc_sh):
    c = lax.axis_index("c")
    s = lax.axis_index("s")
    w = c * NS + s
    nb = jnp.where(w < NBX, NBQ + 1, NBQ)
    _fill(zbuf_v, RPS, 0.0)
    _fill(ones_v, BATCH, 1.0)
    pltpu.sync_copy(zbuf_v, acc_sh.at[pl.ds(s * RPS, RPS)])
    _load_batches(dst_hbm, idx_v, w)
    plsc.subcore_barrier()

    def step(j, _):
        pltpu.sync_copy(ones_v, acc_sh.at[idx_v.at[j]], add=True)
        return 0
    lax.fori_loop(0, nb, step, 0)
    plsc.subcore_barrier()
    pltpu.sync_copy(acc_sh.at[pl.ds(s * RPS, RPS)],
                    out_hbm.at[c, pl.ds(s * RPS, RPS)])


_deg_kernel = pl.kernel(
    _deg_body,
    out_type=jax.ShapeDtypeStruct((NC, NPAD, WPAD), jnp.float32),
    mesh=_mesh,
    compiler_params=pltpu.CompilerParams(use_tc_tiling_on_sc=False),
    scratch_types=[
        pltpu.VMEM((NB, BATCH), jnp.int32),
        pltpu.VMEM((BATCH, WPAD), jnp.float32),
        pltpu.VMEM((RPS, WPAD), jnp.float32),
        pltpu.VMEM_SHARED((NPAD, WPAD), jnp.float32),
    ],
)


def _layer_body(table_hbm, src_hbm, dst_hbm, out_hbm,
                sidx_v, didx_v, rows0_v, rows1_v, rows2_v, rows3_v,
                zbuf_v, acc_sh, tbl_sh, sem0, sem1, sem2, sem3):
    c = lax.axis_index("c")
    s = lax.axis_index("s")
    w = c * NS + s
    nb = jnp.where(w < NBX, NBQ + 1, NBQ)
    _fill(zbuf_v, RPS, 0.0)
    pltpu.sync_copy(zbuf_v, acc_sh.at[pl.ds(s * RPS, RPS)])
    pltpu.sync_copy(table_hbm.at[pl.ds(s * RPS, RPS)],
                    tbl_sh.at[pl.ds(s * RPS, RPS)])
    _zero_idx_row(sidx_v, NBQ)
    _zero_idx_row(sidx_v, NBQ + 1)
    _load_batches(src_hbm, sidx_v, w)
    _load_batches(dst_hbm, didx_v, w)
    plsc.subcore_barrier()

    bufs = (rows0_v, rows1_v, rows2_v, rows3_v)
    sems = (sem0, sem1, sem2, sem3)
    for b in range(NBUF):
        pltpu.async_copy(tbl_sh.at[sidx_v.at[b]], bufs[b], sems[b])
    dummy = table_hbm.at[pl.ds(0, BATCH)]

    def step(i, _):
        j = NBUF * i
        for b in range(NBUF):
            pltpu.make_async_copy(dummy, bufs[b], sems[b]).wait()
            pltpu.sync_copy(bufs[b], acc_sh.at[didx_v.at[j + b]], add=True)
            pltpu.async_copy(tbl_sh.at[sidx_v.at[j + NBUF + b]],
                             bufs[b], sems[b])
        return 0
    lax.fori_loop(0, MAINB // NBUF, step, 0)

    for b in range(NBUF):
        pltpu.make_async_copy(dummy, bufs[b], sems[b]).wait()

        @pl.when(MAINB + b < nb)
        def _(b=b):
            pltpu.sync_copy(bufs[b], acc_sh.at[didx_v.at[MAINB + b]],
                            add=True)

    plsc.subcore_barrier()
    pltpu.sync_copy(acc_sh.at[pl.ds(s * RPS, RPS)],
                    out_hbm.at[c, pl.ds(s * RPS, RPS)])


_layer_kernel = pl.kernel(
    _layer_body,
    out_type=jax.ShapeDtypeStruct((NC, NPAD, WPAD), jnp.float32),
    mesh=_mesh,
    compiler_params=pltpu.CompilerParams(use_tc_tiling_on_sc=False),
    scratch_types=[
        pltpu.VMEM((NB, BATCH), jnp.int32),
        pltpu.VMEM((NB, BATCH), jnp.int32),
        pltpu.VMEM((BATCH, WPAD), jnp.float32),
        pltpu.VMEM((BATCH, WPAD), jnp.float32),
        pltpu.VMEM((BATCH, WPAD), jnp.float32),
        pltpu.VMEM((BATCH, WPAD), jnp.float32),
        pltpu.VMEM((RPS, WPAD), jnp.float32),
        pltpu.VMEM_SHARED((NPAD, WPAD), jnp.float32),
        pltpu.VMEM_SHARED((NPAD, WPAD), jnp.float32),
        pltpu.SemaphoreType.DMA,
        pltpu.SemaphoreType.DMA,
        pltpu.SemaphoreType.DMA,
        pltpu.SemaphoreType.DMA,
    ],
)


def _stage1_tc(x_ref, w1_ref, cnt_ref, y1_ref, dis_ref):
    cnt = cnt_ref[0, 0:N, 0:1] + cnt_ref[1, 0:N, 0:1]
    dis = lax.rsqrt(cnt + 1.0)
    xw = jnp.dot(x_ref[...], w1_ref[...], preferred_element_type=jnp.float32)
    y1_ref[0:N, :] = xw * dis
    y1_ref[N:NPAD, :] = jnp.zeros((NPAD - N, WPAD), jnp.float32)
    dis_ref[...] = dis


def _stage2_tc(p_ref, y1_ref, dis_ref, w2_ref, b1_ref, y2_ref):
    dis = dis_ref[...]
    agg = p_ref[0, 0:N, :] + p_ref[1, 0:N, :] + y1_ref[0:N, :]
    h = jnp.maximum(agg * dis + b1_ref[...], 0.0)
    hw = jnp.dot(h, w2_ref[...], preferred_element_type=jnp.float32)
    y2_ref[0:N, :] = hw * dis
    y2_ref[N:NPAD, :] = jnp.zeros((NPAD - N, WPAD), jnp.float32)


def _stage3_tc(q_ref, y2_ref, dis_ref, b2_ref, out_ref):
    o = (q_ref[0, 0:N, :] + q_ref[1, 0:N, :] + y2_ref[0:N, :]) * dis_ref[...] \
        + b2_ref[...]
    e = jnp.exp(o)
    denom = jnp.maximum(jnp.sum(e, axis=-1, keepdims=True), 1e-12)
    out_ref[...] = e / denom


_stage1 = pl.pallas_call(
    _stage1_tc,
    out_shape=(jax.ShapeDtypeStruct((NPAD, WPAD), jnp.float32),
               jax.ShapeDtypeStruct((N, 1), jnp.float32)),
)

_stage2 = pl.pallas_call(
    _stage2_tc,
    out_shape=jax.ShapeDtypeStruct((NPAD, WPAD), jnp.float32),
)

_stage3 = pl.pallas_call(
    _stage3_tc,
    out_shape=jax.ShapeDtypeStruct((N, WPAD), jnp.float32),
)


def kernel(x, edge_index, W1, b1, W2, b2):
    ei = edge_index.astype(jnp.int32)
    src = ei[0].reshape(NROWS, BATCH)
    dst = ei[1].reshape(NROWS, BATCH)
    w1p = jnp.pad(W1, ((0, 0), (0, WPAD - W1.shape[1])))
    b1p = jnp.pad(b1, (0, WPAD - b1.shape[0])).reshape(1, WPAD)
    w2p = jnp.pad(W2, ((0, WPAD - W2.shape[0]), (0, 0)))
    b2r = b2.reshape(1, WPAD)

    cnt = _deg_kernel(dst)
    y1, dis = _stage1(x, w1p, cnt)
    p = _layer_kernel(y1, src, dst)
    y2 = _stage2(p, y1, dis, w2p, b1p)
    q = _layer_kernel(y2, src, dst)
    return _stage3(q, y2, dis, b2r)


# speedup vs baseline: 1.0263x; 1.0263x over previous
import jax
import jax.numpy as jnp
from jax import lax
from jax.experimental import pallas as pl
from jax.experimental.pallas import tpu as pltpu
from jax.experimental.pallas import tpu_sc as plsc

N = 10000
E = 320000
WPAD = 16
NC, NS = 2, 16
NW = NC * NS
BATCH = 128
NB = 80
NBUF = 4
EPT = NB * BATCH
EPAD = NW * EPT
NPAD = N + 112
RPS = NPAD // NS

_mesh = plsc.VectorSubcoreMesh(core_axis_name="c", subcore_axis_name="s",
                               num_cores=NC, num_subcores=NS)


def _fill(ref, n, val):
    def body(i, _):
        ref[i] = jnp.full((WPAD,), val, jnp.float32)
        return 0
    lax.fori_loop(0, n, body, 0)


def _deg_body(dst_hbm, out_hbm, idx_v, ones_v, zbuf_v, acc_sh):
    c = lax.axis_index("c")
    s = lax.axis_index("s")
    w = c * NS + s
    _fill(zbuf_v, RPS, 0.0)
    _fill(ones_v, BATCH, 1.0)
    pltpu.sync_copy(zbuf_v, acc_sh.at[pl.ds(s * RPS, RPS)])
    pltpu.sync_copy(dst_hbm.at[w], idx_v)
    plsc.subcore_barrier()

    def step(j, _):
        pltpu.sync_copy(ones_v, acc_sh.at[idx_v.at[j]], add=True)
        return 0
    lax.fori_loop(0, NB, step, 0)
    plsc.subcore_barrier()
    pltpu.sync_copy(acc_sh.at[pl.ds(s * RPS, RPS)],
                    out_hbm.at[c, pl.ds(s * RPS, RPS)])


_deg_kernel = pl.kernel(
    _deg_body,
    out_type=jax.ShapeDtypeStruct((NC, NPAD, WPAD), jnp.float32),
    mesh=_mesh,
    compiler_params=pltpu.CompilerParams(use_tc_tiling_on_sc=False),
    scratch_types=[
        pltpu.VMEM((NB, BATCH), jnp.int32),
        pltpu.VMEM((BATCH, WPAD), jnp.float32),
        pltpu.VMEM((RPS, WPAD), jnp.float32),
        pltpu.VMEM_SHARED((NPAD, WPAD), jnp.float32),
    ],
)


def _layer_body(table_hbm, src_hbm, dst_hbm, out_hbm,
                sidx_v, didx_v, rows0_v, rows1_v, rows2_v, rows3_v,
                zbuf_v, acc_sh, tbl_sh, sem0, sem1, sem2, sem3):
    c = lax.axis_index("c")
    s = lax.axis_index("s")
    w = c * NS + s
    _fill(zbuf_v, RPS, 0.0)
    pltpu.sync_copy(zbuf_v, acc_sh.at[pl.ds(s * RPS, RPS)])
    pltpu.sync_copy(table_hbm.at[pl.ds(s * RPS, RPS)],
                    tbl_sh.at[pl.ds(s * RPS, RPS)])
    pltpu.sync_copy(src_hbm.at[w], sidx_v)
    pltpu.sync_copy(dst_hbm.at[w], didx_v)
    plsc.subcore_barrier()

    bufs = (rows0_v, rows1_v, rows2_v, rows3_v)
    sems = (sem0, sem1, sem2, sem3)
    for b in range(NBUF):
        pltpu.async_copy(tbl_sh.at[sidx_v.at[b]], bufs[b], sems[b])
    dummy = table_hbm.at[pl.ds(0, BATCH)]

    def step(i, _):
        j = NBUF * i
        for b in range(NBUF):
            pltpu.make_async_copy(dummy, bufs[b], sems[b]).wait()
            pltpu.sync_copy(bufs[b], acc_sh.at[didx_v.at[j + b]], add=True)
            pltpu.async_copy(tbl_sh.at[sidx_v.at[j + NBUF + b]],
                             bufs[b], sems[b])
        return 0
    lax.fori_loop(0, NB // NBUF - 1, step, 0)

    for b in range(NBUF):
        pltpu.make_async_copy(dummy, bufs[b], sems[b]).wait()
        pltpu.sync_copy(bufs[b], acc_sh.at[didx_v.at[NB - NBUF + b]],
                        add=True)

    plsc.subcore_barrier()
    pltpu.sync_copy(acc_sh.at[pl.ds(s * RPS, RPS)],
                    out_hbm.at[c, pl.ds(s * RPS, RPS)])


_layer_kernel = pl.kernel(
    _layer_body,
    out_type=jax.ShapeDtypeStruct((NC, NPAD, WPAD), jnp.float32),
    mesh=_mesh,
    compiler_params=pltpu.CompilerParams(use_tc_tiling_on_sc=False),
    scratch_types=[
        pltpu.VMEM((NB, BATCH), jnp.int32),
        pltpu.VMEM((NB, BATCH), jnp.int32),
        pltpu.VMEM((BATCH, WPAD), jnp.float32),
        pltpu.VMEM((BATCH, WPAD), jnp.float32),
        pltpu.VMEM((BATCH, WPAD), jnp.float32),
        pltpu.VMEM((BATCH, WPAD), jnp.float32),
        pltpu.VMEM((RPS, WPAD), jnp.float32),
        pltpu.VMEM_SHARED((NPAD, WPAD), jnp.float32),
        pltpu.VMEM_SHARED((NPAD, WPAD), jnp.float32),
        pltpu.SemaphoreType.DMA,
        pltpu.SemaphoreType.DMA,
        pltpu.SemaphoreType.DMA,
        pltpu.SemaphoreType.DMA,
    ],
)


def _stage1_tc(x_ref, w1_ref, cnt_ref, y1_ref, dis_ref):
    cnt = cnt_ref[0, 0:N, 0:1] + cnt_ref[1, 0:N, 0:1]
    dis = lax.rsqrt(cnt + 1.0)
    xw = jnp.dot(x_ref[...], w1_ref[...], preferred_element_type=jnp.float32)
    y1_ref[0:N, :] = xw * dis
    y1_ref[N:NPAD, :] = jnp.zeros((NPAD - N, WPAD), jnp.float32)
    dis_ref[...] = dis


def _stage2_tc(p_ref, y1_ref, dis_ref, w2_ref, b1_ref, y2_ref):
    dis = dis_ref[...]
    agg = p_ref[0, 0:N, :] + p_ref[1, 0:N, :] + y1_ref[0:N, :]
    h = jnp.maximum(agg * dis + b1_ref[...], 0.0)
    hw = jnp.dot(h, w2_ref[...], preferred_element_type=jnp.float32)
    y2_ref[0:N, :] = hw * dis
    y2_ref[N:NPAD, :] = jnp.zeros((NPAD - N, WPAD), jnp.float32)


def _stage3_tc(q_ref, y2_ref, dis_ref, b2_ref, out_ref):
    o = (q_ref[0, 0:N, :] + q_ref[1, 0:N, :] + y2_ref[0:N, :]) * dis_ref[...] \
        + b2_ref[...]
    e = jnp.exp(o)
    denom = jnp.maximum(jnp.sum(e, axis=-1, keepdims=True), 1e-12)
    out_ref[...] = e / denom


_stage1 = pl.pallas_call(
    _stage1_tc,
    out_shape=(jax.ShapeDtypeStruct((NPAD, WPAD), jnp.float32),
               jax.ShapeDtypeStruct((N, 1), jnp.float32)),
)

_stage2 = pl.pallas_call(
    _stage2_tc,
    out_shape=jax.ShapeDtypeStruct((NPAD, WPAD), jnp.float32),
)

_stage3 = pl.pallas_call(
    _stage3_tc,
    out_shape=jax.ShapeDtypeStruct((N, WPAD), jnp.float32),
)


def kernel(x, edge_index, W1, b1, W2, b2):
    ei = edge_index.astype(jnp.int32)
    npe = EPAD - E
    pad_src = jnp.zeros((npe,), jnp.int32)
    pad_dst = N + (jnp.arange(npe, dtype=jnp.int32) % (NPAD - N))
    src = jnp.concatenate([ei[0], pad_src]).reshape(NW, NB, BATCH)
    dst = jnp.concatenate([ei[1], pad_dst]).reshape(NW, NB, BATCH)
    w1p = jnp.pad(W1, ((0, 0), (0, WPAD - W1.shape[1])))
    b1p = jnp.pad(b1, (0, WPAD - b1.shape[0])).reshape(1, WPAD)
    w2p = jnp.pad(W2, ((0, WPAD - W2.shape[0]), (0, 0)))
    b2r = b2.reshape(1, WPAD)

    cnt = _deg_kernel(dst)
    y1, dis = _stage1(x, w1p, cnt)
    p = _layer_kernel(y1, src, dst)
    y2 = _stage2(p, y1, dis, w2p, b1p)
    q = _layer_kernel(y2, src, dst)
    return _stage3(q, y2, dis, b2r)


# speedup vs baseline: 1.0382x; 1.0116x over previous
import jax
import jax.numpy as jnp
from jax import lax
from jax.experimental import pallas as pl
from jax.experimental.pallas import tpu as pltpu
from jax.experimental.pallas import tpu_sc as plsc

N = 10000
E = 320000
WPAD = 16
NC, NS = 2, 16
NW = NC * NS
BATCH = 128
NB = 80
NBUF = 4
EPT = NB * BATCH
EPAD = NW * EPT
NPAD = N + 112
RPS = NPAD // NS

_mesh = plsc.VectorSubcoreMesh(core_axis_name="c", subcore_axis_name="s",
                               num_cores=NC, num_subcores=NS)


def _fill(ref, n, val):
    def body(i, _):
        ref[i] = jnp.full((WPAD,), val, jnp.float32)
        return 0
    lax.fori_loop(0, n, body, 0)


WDEG = 8


def _deg_body(dst_hbm, ones_hbm, zeros_hbm, out_hbm, idx_v, ones_v, acc_sh):
    c = lax.axis_index("c")
    s = lax.axis_index("s")
    w = c * NS + s
    pltpu.sync_copy(ones_hbm, ones_v)
    pltpu.sync_copy(zeros_hbm, acc_sh.at[pl.ds(s * RPS, RPS)])
    pltpu.sync_copy(dst_hbm.at[w], idx_v)
    plsc.subcore_barrier()

    def step(j, _):
        pltpu.sync_copy(ones_v, acc_sh.at[idx_v.at[j]], add=True)
        return 0
    lax.fori_loop(0, NB, step, 0)
    plsc.subcore_barrier()
    pltpu.sync_copy(acc_sh.at[pl.ds(s * RPS, RPS)],
                    out_hbm.at[c, pl.ds(s * RPS, RPS)])


_deg_kernel = pl.kernel(
    _deg_body,
    out_type=jax.ShapeDtypeStruct((NC, NPAD, WDEG), jnp.float32),
    mesh=_mesh,
    compiler_params=pltpu.CompilerParams(use_tc_tiling_on_sc=False),
    scratch_types=[
        pltpu.VMEM((NB, BATCH), jnp.int32),
        pltpu.VMEM((BATCH, WDEG), jnp.float32),
        pltpu.VMEM_SHARED((NPAD, WDEG), jnp.float32),
    ],
)


def _layer_body(table_hbm, src_hbm, dst_hbm, out_hbm,
                sidx_v, didx_v, rows0_v, rows1_v, rows2_v, rows3_v,
                zbuf_v, acc_sh, tbl_sh, sem0, sem1, sem2, sem3):
    c = lax.axis_index("c")
    s = lax.axis_index("s")
    w = c * NS + s
    _fill(zbuf_v, RPS, 0.0)
    pltpu.sync_copy(zbuf_v, acc_sh.at[pl.ds(s * RPS, RPS)])
    pltpu.sync_copy(table_hbm.at[pl.ds(s * RPS, RPS)],
                    tbl_sh.at[pl.ds(s * RPS, RPS)])
    pltpu.sync_copy(src_hbm.at[w], sidx_v)
    pltpu.sync_copy(dst_hbm.at[w], didx_v)
    plsc.subcore_barrier()

    bufs = (rows0_v, rows1_v, rows2_v, rows3_v)
    sems = (sem0, sem1, sem2, sem3)
    for b in range(NBUF):
        pltpu.async_copy(tbl_sh.at[sidx_v.at[b]], bufs[b], sems[b])
    dummy = table_hbm.at[pl.ds(0, BATCH)]

    def step(i, _):
        j = NBUF * i
        for b in range(NBUF):
            pltpu.make_async_copy(dummy, bufs[b], sems[b]).wait()
            pltpu.sync_copy(bufs[b], acc_sh.at[didx_v.at[j + b]], add=True)
            pltpu.async_copy(tbl_sh.at[sidx_v.at[j + NBUF + b]],
                             bufs[b], sems[b])
        return 0
    lax.fori_loop(0, NB // NBUF - 1, step, 0)

    for b in range(NBUF):
        pltpu.make_async_copy(dummy, bufs[b], sems[b]).wait()
        pltpu.sync_copy(bufs[b], acc_sh.at[didx_v.at[NB - NBUF + b]],
                        add=True)

    plsc.subcore_barrier()
    pltpu.sync_copy(acc_sh.at[pl.ds(s * RPS, RPS)],
                    out_hbm.at[c, pl.ds(s * RPS, RPS)])


_layer_kernel = pl.kernel(
    _layer_body,
    out_type=jax.ShapeDtypeStruct((NC, NPAD, WPAD), jnp.float32),
    mesh=_mesh,
    compiler_params=pltpu.CompilerParams(use_tc_tiling_on_sc=False),
    scratch_types=[
        pltpu.VMEM((NB, BATCH), jnp.int32),
        pltpu.VMEM((NB, BATCH), jnp.int32),
        pltpu.VMEM((BATCH, WPAD), jnp.float32),
        pltpu.VMEM((BATCH, WPAD), jnp.float32),
        pltpu.VMEM((BATCH, WPAD), jnp.float32),
        pltpu.VMEM((BATCH, WPAD), jnp.float32),
        pltpu.VMEM((RPS, WPAD), jnp.float32),
        pltpu.VMEM_SHARED((NPAD, WPAD), jnp.float32),
        pltpu.VMEM_SHARED((NPAD, WPAD), jnp.float32),
        pltpu.SemaphoreType.DMA,
        pltpu.SemaphoreType.DMA,
        pltpu.SemaphoreType.DMA,
        pltpu.SemaphoreType.DMA,
    ],
)


def _stage1_tc(x_ref, w1_ref, cnt_ref, y1_ref, dis_ref):
    cnt = cnt_ref[0, 0:N, 0:1] + cnt_ref[1, 0:N, 0:1]
    dis = lax.rsqrt(cnt + 1.0)
    xw = jnp.dot(x_ref[...], w1_ref[...], preferred_element_type=jnp.float32)
    y1_ref[0:N, :] = xw * dis
    y1_ref[N:NPAD, :] = jnp.zeros((NPAD - N, WPAD), jnp.float32)
    dis_ref[...] = dis


def _stage2_tc(p_ref, y1_ref, dis_ref, w2_ref, b1_ref, y2_ref):
    dis = dis_ref[...]
    agg = p_ref[0, 0:N, :] + p_ref[1, 0:N, :] + y1_ref[0:N, :]
    h = jnp.maximum(agg * dis + b1_ref[...], 0.0)
    hw = jnp.dot(h, w2_ref[...], preferred_element_type=jnp.float32)
    y2_ref[0:N, :] = hw * dis
    y2_ref[N:NPAD, :] = jnp.zeros((NPAD - N, WPAD), jnp.float32)


def _stage3_tc(q_ref, y2_ref, dis_ref, b2_ref, out_ref):
    o = (q_ref[0, 0:N, :] + q_ref[1, 0:N, :] + y2_ref[0:N, :]) * dis_ref[...] \
        + b2_ref[...]
    e = jnp.exp(o)
    denom = jnp.maximum(jnp.sum(e, axis=-1, keepdims=True), 1e-12)
    out_ref[...] = e / denom


_stage1 = pl.pallas_call(
    _stage1_tc,
    out_shape=(jax.ShapeDtypeStruct((NPAD, WPAD), jnp.float32),
               jax.ShapeDtypeStruct((N, 1), jnp.float32)),
)

_stage2 = pl.pallas_call(
    _stage2_tc,
    out_shape=jax.ShapeDtypeStruct((NPAD, WPAD), jnp.float32),
)

_stage3 = pl.pallas_call(
    _stage3_tc,
    out_shape=jax.ShapeDtypeStruct((N, WPAD), jnp.float32),
)


def kernel(x, edge_index, W1, b1, W2, b2):
    ei = edge_index.astype(jnp.int32)
    npe = EPAD - E
    pad_src = jnp.zeros((npe,), jnp.int32)
    pad_dst = N + (jnp.arange(npe, dtype=jnp.int32) % (NPAD - N))
    src = jnp.concatenate([ei[0], pad_src]).reshape(NW, NB, BATCH)
    dst = jnp.concatenate([ei[1], pad_dst]).reshape(NW, NB, BATCH)
    w1p = jnp.pad(W1, ((0, 0), (0, WPAD - W1.shape[1])))
    b1p = jnp.pad(b1, (0, WPAD - b1.shape[0])).reshape(1, WPAD)
    w2p = jnp.pad(W2, ((0, WPAD - W2.shape[0]), (0, 0)))
    b2r = b2.reshape(1, WPAD)
    ones8 = jnp.ones((BATCH, WDEG), jnp.float32)
    zeros8 = jnp.zeros((RPS, WDEG), jnp.float32)

    cnt = _deg_kernel(dst, ones8, zeros8)
    y1, dis = _stage1(x, w1p, cnt)
    p = _layer_kernel(y1, src, dst)
    y2 = _stage2(p, y1, dis, w2p, b1p)
    q = _layer_kernel(y2, src, dst)
    return _stage3(q, y2, dis, b2r)


# speedup vs baseline: 1.0850x; 1.0451x over previous
import jax
import jax.numpy as jnp
from jax import lax
from jax.experimental import pallas as pl
from jax.experimental.pallas import tpu as pltpu
from jax.experimental.pallas import tpu_sc as plsc

N = 10000
E = 320000
WPAD = 16
NC, NS = 2, 16
NW = NC * NS
BATCH = 128
NB = 80
NBUF = 4
EPT = NB * BATCH
EPAD = NW * EPT
NPAD = N + 112
RPS = NPAD // NS

_mesh = plsc.VectorSubcoreMesh(core_axis_name="c", subcore_axis_name="s",
                               num_cores=NC, num_subcores=NS)


def _fill(ref, n, val):
    def body(i, _):
        ref[i] = jnp.full((WPAD,), val, jnp.float32)
        return 0
    lax.fori_loop(0, n, body, 0)


WDEG = 8


def _deg_body(dst_hbm, ones_hbm, zeros_hbm, out_hbm, idx_v, ones_v, acc_sh):
    c = lax.axis_index("c")
    s = lax.axis_index("s")
    w = c * NS + s
    pltpu.sync_copy(ones_hbm, ones_v)
    pltpu.sync_copy(zeros_hbm, acc_sh.at[pl.ds(s * RPS, RPS)])
    pltpu.sync_copy(dst_hbm.at[w], idx_v)
    plsc.subcore_barrier()

    def step(j, _):
        pltpu.sync_copy(ones_v, acc_sh.at[idx_v.at[j]], add=True)
        return 0
    lax.fori_loop(0, NB, step, 0)
    plsc.subcore_barrier()
    pltpu.sync_copy(acc_sh.at[pl.ds(s * RPS, RPS)],
                    out_hbm.at[c, pl.ds(s * RPS, RPS)])


_deg_kernel = pl.kernel(
    _deg_body,
    out_type=jax.ShapeDtypeStruct((NC, NPAD, WDEG), jnp.float32),
    mesh=_mesh,
    compiler_params=pltpu.CompilerParams(use_tc_tiling_on_sc=False),
    scratch_types=[
        pltpu.VMEM((NB, BATCH), jnp.int32),
        pltpu.VMEM((BATCH, WDEG), jnp.float32),
        pltpu.VMEM_SHARED((NPAD, WDEG), jnp.float32),
    ],
)


def _layer_body(table_hbm, src_hbm, dst_hbm, out_hbm,
                sidx_v, didx_v, rows0_v, rows1_v, rows2_v, rows3_v,
                zbuf_v, acc_sh, tbl_sh, sem0, sem1, sem2, sem3):
    c = lax.axis_index("c")
    s = lax.axis_index("s")
    w = c * NS + s
    cp_t = pltpu.make_async_copy(table_hbm.at[pl.ds(s * RPS, RPS)],
                                 tbl_sh.at[pl.ds(s * RPS, RPS)], sem0)
    cp_s = pltpu.make_async_copy(src_hbm.at[w], sidx_v, sem1)
    cp_d = pltpu.make_async_copy(dst_hbm.at[w], didx_v, sem2)
    cp_t.start()
    cp_s.start()
    cp_d.start()
    _fill(zbuf_v, RPS, 0.0)
    pltpu.sync_copy(zbuf_v, acc_sh.at[pl.ds(s * RPS, RPS)])
    cp_t.wait()
    cp_s.wait()
    cp_d.wait()
    plsc.subcore_barrier()

    bufs = (rows0_v, rows1_v, rows2_v, rows3_v)
    sems = (sem0, sem1, sem2, sem3)
    for b in range(NBUF):
        pltpu.async_copy(tbl_sh.at[sidx_v.at[b]], bufs[b], sems[b])
    dummy = table_hbm.at[pl.ds(0, BATCH)]

    def step(i, _):
        j = NBUF * i
        for b in range(NBUF):
            pltpu.make_async_copy(dummy, bufs[b], sems[b]).wait()
            pltpu.sync_copy(bufs[b], acc_sh.at[didx_v.at[j + b]], add=True)
            pltpu.async_copy(tbl_sh.at[sidx_v.at[j + NBUF + b]],
                             bufs[b], sems[b])
        return 0
    lax.fori_loop(0, NB // NBUF - 1, step, 0)

    for b in range(NBUF):
        pltpu.make_async_copy(dummy, bufs[b], sems[b]).wait()
        pltpu.sync_copy(bufs[b], acc_sh.at[didx_v.at[NB - NBUF + b]],
                        add=True)

    plsc.subcore_barrier()
    pltpu.sync_copy(acc_sh.at[pl.ds(s * RPS, RPS)],
                    out_hbm.at[c, pl.ds(s * RPS, RPS)])


_layer_kernel = pl.kernel(
    _layer_body,
    out_type=jax.ShapeDtypeStruct((NC, NPAD, WPAD), jnp.float32),
    mesh=_mesh,
    compiler_params=pltpu.CompilerParams(use_tc_tiling_on_sc=False),
    scratch_types=[
        pltpu.VMEM((NB, BATCH), jnp.int32),
        pltpu.VMEM((NB, BATCH), jnp.int32),
        pltpu.VMEM((BATCH, WPAD), jnp.float32),
        pltpu.VMEM((BATCH, WPAD), jnp.float32),
        pltpu.VMEM((BATCH, WPAD), jnp.float32),
        pltpu.VMEM((BATCH, WPAD), jnp.float32),
        pltpu.VMEM((RPS, WPAD), jnp.float32),
        pltpu.VMEM_SHARED((NPAD, WPAD), jnp.float32),
        pltpu.VMEM_SHARED((NPAD, WPAD), jnp.float32),
        pltpu.SemaphoreType.DMA,
        pltpu.SemaphoreType.DMA,
        pltpu.SemaphoreType.DMA,
        pltpu.SemaphoreType.DMA,
    ],
)


def _stage1_tc(x_ref, w1_ref, cnt_ref, y1_ref, dis_ref):
    cnt = cnt_ref[0, 0:N, 0:1] + cnt_ref[1, 0:N, 0:1]
    dis = lax.rsqrt(cnt + 1.0)
    xw = jnp.dot(x_ref[...], w1_ref[...], preferred_element_type=jnp.float32)
    y1_ref[0:N, :] = xw * dis
    y1_ref[N:NPAD, :] = jnp.zeros((NPAD - N, WPAD), jnp.float32)
    dis_ref[...] = dis


def _stage2_tc(p_ref, y1_ref, dis_ref, w2_ref, b1_ref, y2_ref):
    dis = dis_ref[...]
    agg = p_ref[0, 0:N, :] + p_ref[1, 0:N, :] + y1_ref[0:N, :]
    h = jnp.maximum(agg * dis + b1_ref[...], 0.0)
    hw = jnp.dot(h, w2_ref[...], preferred_element_type=jnp.float32)
    y2_ref[0:N, :] = hw * dis
    y2_ref[N:NPAD, :] = jnp.zeros((NPAD - N, WPAD), jnp.float32)


def _stage3_tc(q_ref, y2_ref, dis_ref, b2_ref, out_ref):
    o = (q_ref[0, 0:N, :] + q_ref[1, 0:N, :] + y2_ref[0:N, :]) * dis_ref[...] \
        + b2_ref[...]
    e = jnp.exp(o)
    denom = jnp.maximum(jnp.sum(e, axis=-1, keepdims=True), 1e-12)
    out_ref[...] = e / denom


_stage1 = pl.pallas_call(
    _stage1_tc,
    out_shape=(jax.ShapeDtypeStruct((NPAD, WPAD), jnp.float32),
               jax.ShapeDtypeStruct((N, 1), jnp.float32)),
)

_stage2 = pl.pallas_call(
    _stage2_tc,
    out_shape=jax.ShapeDtypeStruct((NPAD, WPAD), jnp.float32),
)

_stage3 = pl.pallas_call(
    _stage3_tc,
    out_shape=jax.ShapeDtypeStruct((N, WPAD), jnp.float32),
)


def kernel(x, edge_index, W1, b1, W2, b2):
    ei = edge_index.astype(jnp.int32)
    npe = EPAD - E
    pad_src = jnp.zeros((npe,), jnp.int32)
    pad_dst = N + (jnp.arange(npe, dtype=jnp.int32) % (NPAD - N))
    src = jnp.concatenate([ei[0], pad_src]).reshape(NW, NB, BATCH)
    dst = jnp.concatenate([ei[1], pad_dst]).reshape(NW, NB, BATCH)
    w1p = jnp.pad(W1, ((0, 0), (0, WPAD - W1.shape[1])))
    b1p = jnp.pad(b1, (0, WPAD - b1.shape[0])).reshape(1, WPAD)
    w2p = jnp.pad(W2, ((0, WPAD - W2.shape[0]), (0, 0)))
    b2r = b2.reshape(1, WPAD)
    ones8 = jnp.ones((BATCH, WDEG), jnp.float32)
    zeros8 = jnp.zeros((RPS, WDEG), jnp.float32)

    cnt = _deg_kernel(dst, ones8, zeros8)
    y1, dis = _stage1(x, w1p, cnt)
    p = _layer_kernel(y1, src, dst)
    y2 = _stage2(p, y1, dis, w2p, b1p)
    q = _layer_kernel(y2, src, dst)
    return _stage3(q, y2, dis, b2r)


# speedup vs baseline: 1.0943x; 1.0086x over previous
import jax
import jax.numpy as jnp
from jax import lax
from jax.experimental import pallas as pl
from jax.experimental.pallas import tpu as pltpu
from jax.experimental.pallas import tpu_sc as plsc

N = 10000
E = 320000
WPAD = 16
NC, NS = 2, 16
NW = NC * NS
BATCH = 128
NB = 80
NBUF = 4
EPT = NB * BATCH
EPAD = NW * EPT
NPAD = N + 112
RPS = NPAD // NS

_mesh = plsc.VectorSubcoreMesh(core_axis_name="c", subcore_axis_name="s",
                               num_cores=NC, num_subcores=NS)


def _fill(ref, n, val):
    def body(i, _):
        ref[i] = jnp.full((WPAD,), val, jnp.float32)
        return 0
    lax.fori_loop(0, n, body, 0)


WDEG = 8


def _deg_body(dst_hbm, ones_hbm, zeros_hbm, out_hbm, idx_v, ones_v, acc_sh,
              sem0, sem1):
    c = lax.axis_index("c")
    s = lax.axis_index("s")
    w = c * NS + s
    cp_o = pltpu.make_async_copy(ones_hbm, ones_v, sem0)
    cp_i = pltpu.make_async_copy(dst_hbm.at[w], idx_v, sem1)
    cp_o.start()
    cp_i.start()
    pltpu.sync_copy(zeros_hbm, acc_sh.at[pl.ds(s * RPS, RPS)])
    cp_o.wait()
    cp_i.wait()
    plsc.subcore_barrier()

    def step(j, _):
        pltpu.sync_copy(ones_v, acc_sh.at[idx_v.at[j]], add=True)
        return 0
    lax.fori_loop(0, NB, step, 0)
    plsc.subcore_barrier()
    pltpu.sync_copy(acc_sh.at[pl.ds(s * RPS, RPS)],
                    out_hbm.at[c, pl.ds(s * RPS, RPS)])


_deg_kernel = pl.kernel(
    _deg_body,
    out_type=jax.ShapeDtypeStruct((NC, NPAD, WDEG), jnp.float32),
    mesh=_mesh,
    compiler_params=pltpu.CompilerParams(use_tc_tiling_on_sc=False),
    scratch_types=[
        pltpu.VMEM((NB, BATCH), jnp.int32),
        pltpu.VMEM((BATCH, WDEG), jnp.float32),
        pltpu.VMEM_SHARED((NPAD, WDEG), jnp.float32),
        pltpu.SemaphoreType.DMA,
        pltpu.SemaphoreType.DMA,
    ],
)


def _layer_body(table_hbm, src_hbm, dst_hbm, out_hbm,
                sidx_v, didx_v, rows0_v, rows1_v, rows2_v, rows3_v,
                zbuf_v, acc_sh, tbl_sh, sem0, sem1, sem2, sem3):
    c = lax.axis_index("c")
    s = lax.axis_index("s")
    w = c * NS + s
    cp_t = pltpu.make_async_copy(table_hbm.at[pl.ds(s * RPS, RPS)],
                                 tbl_sh.at[pl.ds(s * RPS, RPS)], sem0)
    cp_s = pltpu.make_async_copy(src_hbm.at[w], sidx_v, sem1)
    cp_d = pltpu.make_async_copy(dst_hbm.at[w], didx_v, sem2)
    cp_t.start()
    cp_s.start()
    cp_d.start()
    _fill(zbuf_v, RPS, 0.0)
    pltpu.sync_copy(zbuf_v, acc_sh.at[pl.ds(s * RPS, RPS)])
    cp_t.wait()
    cp_s.wait()
    cp_d.wait()
    plsc.subcore_barrier()

    bufs = (rows0_v, rows1_v, rows2_v, rows3_v)
    sems = (sem0, sem1, sem2, sem3)
    for b in range(NBUF):
        pltpu.async_copy(tbl_sh.at[sidx_v.at[b]], bufs[b], sems[b])
    dummy = table_hbm.at[pl.ds(0, BATCH)]

    def step(i, _):
        j = NBUF * i
        for b in range(NBUF):
            pltpu.make_async_copy(dummy, bufs[b], sems[b]).wait()
            pltpu.sync_copy(bufs[b], acc_sh.at[didx_v.at[j + b]], add=True)
            pltpu.async_copy(tbl_sh.at[sidx_v.at[j + NBUF + b]],
                             bufs[b], sems[b])
        return 0
    lax.fori_loop(0, NB // NBUF - 1, step, 0)

    for b in range(NBUF):
        pltpu.make_async_copy(dummy, bufs[b], sems[b]).wait()
        pltpu.sync_copy(bufs[b], acc_sh.at[didx_v.at[NB - NBUF + b]],
                        add=True)

    plsc.subcore_barrier()
    pltpu.sync_copy(acc_sh.at[pl.ds(s * RPS, RPS)],
                    out_hbm.at[c, pl.ds(s * RPS, RPS)])


_layer_kernel = pl.kernel(
    _layer_body,
    out_type=jax.ShapeDtypeStruct((NC, NPAD, WPAD), jnp.float32),
    mesh=_mesh,
    compiler_params=pltpu.CompilerParams(use_tc_tiling_on_sc=False),
    scratch_types=[
        pltpu.VMEM((NB, BATCH), jnp.int32),
        pltpu.VMEM((NB, BATCH), jnp.int32),
        pltpu.VMEM((BATCH, WPAD), jnp.float32),
        pltpu.VMEM((BATCH, WPAD), jnp.float32),
        pltpu.VMEM((BATCH, WPAD), jnp.float32),
        pltpu.VMEM((BATCH, WPAD), jnp.float32),
        pltpu.VMEM((RPS, WPAD), jnp.float32),
        pltpu.VMEM_SHARED((NPAD, WPAD), jnp.float32),
        pltpu.VMEM_SHARED((NPAD, WPAD), jnp.float32),
        pltpu.SemaphoreType.DMA,
        pltpu.SemaphoreType.DMA,
        pltpu.SemaphoreType.DMA,
        pltpu.SemaphoreType.DMA,
    ],
)


def _stage1_tc(x_ref, w1_ref, cnt_ref, y1_ref, dis_ref):
    cnt = cnt_ref[0, 0:N, 0:1] + cnt_ref[1, 0:N, 0:1]
    dis = lax.rsqrt(cnt + 1.0)
    xw = jnp.dot(x_ref[...], w1_ref[...], preferred_element_type=jnp.float32)
    y1_ref[0:N, :] = xw * dis
    y1_ref[N:NPAD, :] = jnp.zeros((NPAD - N, WPAD), jnp.float32)
    dis_ref[...] = dis


def _stage2_tc(p_ref, y1_ref, dis_ref, w2_ref, b1_ref, y2_ref):
    dis = dis_ref[...]
    agg = p_ref[0, 0:N, :] + p_ref[1, 0:N, :] + y1_ref[0:N, :]
    h = jnp.maximum(agg * dis + b1_ref[...], 0.0)
    hw = jnp.dot(h, w2_ref[...], preferred_element_type=jnp.float32)
    y2_ref[0:N, :] = hw * dis
    y2_ref[N:NPAD, :] = jnp.zeros((NPAD - N, WPAD), jnp.float32)


def _stage3_tc(q_ref, y2_ref, dis_ref, b2_ref, out_ref):
    o = (q_ref[0, 0:N, :] + q_ref[1, 0:N, :] + y2_ref[0:N, :]) * dis_ref[...] \
        + b2_ref[...]
    e = jnp.exp(o)
    denom = jnp.maximum(jnp.sum(e, axis=-1, keepdims=True), 1e-12)
    out_ref[...] = e / denom


_stage1 = pl.pallas_call(
    _stage1_tc,
    out_shape=(jax.ShapeDtypeStruct((NPAD, WPAD), jnp.float32),
               jax.ShapeDtypeStruct((N, 1), jnp.float32)),
)

_stage2 = pl.pallas_call(
    _stage2_tc,
    out_shape=jax.ShapeDtypeStruct((NPAD, WPAD), jnp.float32),
)

_stage3 = pl.pallas_call(
    _stage3_tc,
    out_shape=jax.ShapeDtypeStruct((N, WPAD), jnp.float32),
)


def kernel(x, edge_index, W1, b1, W2, b2):
    ei = edge_index.astype(jnp.int32)
    npe = EPAD - E
    pad_src = jnp.zeros((npe,), jnp.int32)
    pad_dst = N + (jnp.arange(npe, dtype=jnp.int32) % (NPAD - N))
    src = jnp.concatenate([ei[0], pad_src]).reshape(NW, NB, BATCH)
    dst = jnp.concatenate([ei[1], pad_dst]).reshape(NW, NB, BATCH)
    w1p = jnp.pad(W1, ((0, 0), (0, WPAD - W1.shape[1])))
    b1p = jnp.pad(b1, (0, WPAD - b1.shape[0])).reshape(1, WPAD)
    w2p = jnp.pad(W2, ((0, WPAD - W2.shape[0]), (0, 0)))
    b2r = b2.reshape(1, WPAD)
    ones8 = jnp.ones((BATCH, WDEG), jnp.float32)
    zeros8 = jnp.zeros((RPS, WDEG), jnp.float32)

    cnt = _deg_kernel(dst, ones8, zeros8)
    y1, dis = _stage1(x, w1p, cnt)
    p = _layer_kernel(y1, src, dst)
    y2 = _stage2(p, y1, dis, w2p, b1p)
    q = _layer_kernel(y2, src, dst)
    return _stage3(q, y2, dis, b2r)
